# Initial kernel scaffold; baseline (speedup 1.0000x reference)
import functools

import jax
import jax.numpy as jnp
from jax import lax
from jax.experimental import pallas as pl
from jax.experimental.pallas import tpu as pltpu
from jax.experimental.pallas import tpu_sc as plsc

N = 10000
NP = 10240
E = 320000
CH = 128
NW = 32
NCHUNK = 79
EPW = NCHUNK * CH
EP = NW * EPW
NSUB = 16
RPS = NP // NSUB

D1 = 16
D2 = 48
NCLS = 40


def _sc_agg_kernel(D):
    mesh = plsc.VectorSubcoreMesh(core_axis_name="c", subcore_axis_name="s")

    @functools.partial(
        pl.kernel,
        out_type=jax.ShapeDtypeStruct((2, NP, D), jnp.float32),
        mesh=mesh,
        scratch_types=[
            pltpu.VMEM((CH,), jnp.int32),
            pltpu.VMEM((CH,), jnp.int32),
            pltpu.VMEM((CH, D), jnp.float32),
            pltpu.VMEM_SHARED((NP, D), jnp.float32),
            pltpu.SemaphoreType.DMA,
        ],
    )
    def kernel(y_hbm, src_hbm, dst_hbm, out_hbm, sidx, didx, rows, acc, sem):
        cid = lax.axis_index("c")
        sid = lax.axis_index("s")
        wid = sid * 2 + cid
        base = wid * EPW

        @pl.loop(0, CH)
        def _(r):
            @pl.loop(0, D, step=16)
            def _(c2):
                rows[r, pl.ds(c2, 16)] = jnp.zeros((16,), jnp.float32)

        @pl.loop(0, RPS, step=CH)
        def _(j):
            pltpu.sync_copy(rows, acc.at[pl.ds(sid * RPS + j, CH)])

        plsc.subcore_barrier()

        @pl.loop(0, NCHUNK)
        def _(i):
            off = base + i * CH
            pltpu.sync_copy(src_hbm.at[pl.ds(off, CH)], sidx)
            pltpu.sync_copy(dst_hbm.at[pl.ds(off, CH)], didx)
            pltpu.async_copy(y_hbm.at[sidx], rows, sem).wait()
            pltpu.sync_copy(rows, acc.at[didx], add=True)

        plsc.subcore_barrier()
        pltpu.sync_copy(
            acc.at[pl.ds(sid * RPS, RPS)],
            out_hbm.at[cid, pl.ds(sid * RPS, RPS)],
        )

    return kernel


def _sc_deg_kernel():
    mesh = plsc.VectorSubcoreMesh(core_axis_name="c", subcore_axis_name="s")

    @functools.partial(
        pl.kernel,
        out_type=jax.ShapeDtypeStruct((2, NP, 16), jnp.float32),
        mesh=mesh,
        scratch_types=[
            pltpu.VMEM((CH,), jnp.int32),
            pltpu.VMEM((CH, 16), jnp.float32),
            pltpu.VMEM_SHARED((NP, 16), jnp.float32),
            pltpu.SemaphoreType.DMA,
        ],
    )
    def kernel(dst_hbm, out_hbm, didx, vals, acc, sem):
        cid = lax.axis_index("c")
        sid = lax.axis_index("s")
        wid = sid * 2 + cid
        base = wid * EPW

        @pl.loop(0, CH)
        def _(r):
            vals[r, pl.ds(0, 16)] = jnp.zeros((16,), jnp.float32)

        @pl.loop(0, RPS, step=CH)
        def _(j):
            pltpu.sync_copy(vals, acc.at[pl.ds(sid * RPS + j, CH)])

        @pl.loop(0, CH)
        def _(r):
            vals[r, pl.ds(0, 16)] = jnp.full((16,), 1.0, jnp.float32)

        plsc.subcore_barrier()

        @pl.loop(0, NCHUNK)
        def _(i):
            off = base + i * CH
            pltpu.sync_copy(dst_hbm.at[pl.ds(off, CH)], didx)
            pltpu.sync_copy(vals, acc.at[didx], add=True)

        plsc.subcore_barrier()
        pltpu.sync_copy(
            acc.at[pl.ds(sid * RPS, RPS)],
            out_hbm.at[cid, pl.ds(sid * RPS, RPS)],
        )

    return kernel


def _tc_layer1(x_pad, W1, degacc):
    def body(x_ref, w_ref, d_ref, y1_ref, dinv_ref):
        d = d_ref[...]
        deg = d[0, :, 0:1] + d[1, :, 0:1] + 1.0
        dinv = lax.rsqrt(deg)
        xw = jnp.dot(x_ref[...], w_ref[...], preferred_element_type=jnp.float32)
        y1_ref[...] = xw * dinv
        dinv_ref[...] = dinv

    return pl.pallas_call(
        body,
        out_shape=(
            jax.ShapeDtypeStruct((NP, D1), jnp.float32),
            jax.ShapeDtypeStruct((NP, 1), jnp.float32),
        ),
    )(x_pad, W1, degacc)


def _tc_layer2(agg1, y1, dinv, b1, W2p):
    def body(a_ref, y1_ref, dinv_ref, b1_ref, w2_ref, y2_ref):
        a = a_ref[...]
        dinv = dinv_ref[...]
        s = (a[0] + a[1] + y1_ref[...]) * dinv + b1_ref[...]
        h = jnp.maximum(s, 0.0)
        y2 = jnp.dot(h, w2_ref[...], preferred_element_type=jnp.float32) * dinv
        row = lax.broadcasted_iota(jnp.int32, (NP, D2), 0)
        y2_ref[...] = jnp.where(row < N, y2, 0.0)

    return pl.pallas_call(
        body,
        out_shape=jax.ShapeDtypeStruct((NP, D2), jnp.float32),
    )(agg1, y1, dinv, b1, W2p)


def _tc_final(agg2, y2, dinv, b2):
    def body(a_ref, y2_ref, dinv_ref, b2_ref, o_ref):
        a = a_ref[...]
        s = (a[0] + a[1] + y2_ref[...]) * dinv_ref[...]
        o = s[:N, :NCLS] + b2_ref[...]
        m = jnp.max(o, axis=1, keepdims=True)
        e = jnp.exp(o - m)
        lse = jnp.log(jnp.sum(e, axis=1, keepdims=True))
        o_ref[...] = o - m - lse

    return pl.pallas_call(
        body,
        out_shape=jax.ShapeDtypeStruct((N, NCLS), jnp.float32),
    )(agg2, y2, dinv, b2)


def kernel(x, edge_index, W1, b1, W2, b2):
    src = edge_index[0].astype(jnp.int32)
    dst = edge_index[1].astype(jnp.int32)
    npad = EP - E
    src_p = jnp.concatenate([src, jnp.full((npad,), N, jnp.int32)])
    dst_p = jnp.concatenate(
        [dst, N + (jnp.arange(npad, dtype=jnp.int32) % (NP - N))]
    )
    x_pad = jnp.pad(x, ((0, NP - N), (0, 0)))
    b1r = b1.reshape(1, D1)
    W2p = jnp.pad(W2, ((0, 0), (0, D2 - NCLS)))
    b2r = b2.reshape(1, NCLS)

    degacc = _sc_deg_kernel()(dst_p)
    y1, dinv = _tc_layer1(x_pad, W1, degacc)
    agg1 = _sc_agg_kernel(D1)(y1, src_p, dst_p)
    y2 = _tc_layer2(agg1, y1, dinv, b1r, W2p)
    agg2 = _sc_agg_kernel(D2)(y2, src_p, dst_p)
    return _tc_final(agg2, y2, dinv, b2r)


# speedup vs baseline: 18.5175x; 18.5175x over previous
import functools

import jax
import jax.numpy as jnp
from jax import lax
from jax.experimental import pallas as pl
from jax.experimental.pallas import tpu as pltpu
from jax.experimental.pallas import tpu_sc as plsc

_SC_PARAMS = pltpu.CompilerParams(use_tc_tiling_on_sc=False)

N = 10000
NP = 10240
E = 320000
CH = 128
NW = 32
NCHUNK = 79
EPW = NCHUNK * CH
EP = NW * EPW
NSUB = 16
RPS = NP // NSUB

D1 = 16
D2 = 48
NCLS = 40


def _sc_agg_kernel(D):
    mesh = plsc.VectorSubcoreMesh(core_axis_name="c", subcore_axis_name="s")

    @functools.partial(
        pl.kernel,
        out_type=jax.ShapeDtypeStruct((2, NP, D), jnp.float32),
        mesh=mesh,
        scratch_types=[
            pltpu.VMEM((CH,), jnp.int32),
            pltpu.VMEM((CH,), jnp.int32),
            pltpu.VMEM((CH, D), jnp.float32),
            pltpu.VMEM_SHARED((NP, D), jnp.float32),
            pltpu.SemaphoreType.DMA,
        ],
        compiler_params=_SC_PARAMS,
    )
    def kernel(y_hbm, src_hbm, dst_hbm, out_hbm, sidx, didx, rows, acc, sem):
        cid = lax.axis_index("c")
        sid = lax.axis_index("s")
        wid = sid * 2 + cid
        base = wid * EPW

        @pl.loop(0, CH)
        def _(r):
            @pl.loop(0, D, step=16)
            def _(c2):
                rows[r, pl.ds(c2, 16)] = jnp.zeros((16,), jnp.float32)

        @pl.loop(0, RPS, step=CH)
        def _(j):
            pltpu.sync_copy(rows, acc.at[pl.ds(sid * RPS + j, CH)])

        plsc.subcore_barrier()

        @pl.loop(0, NCHUNK)
        def _(i):
            off = base + i * CH
            pltpu.sync_copy(src_hbm.at[pl.ds(off, CH)], sidx)
            pltpu.sync_copy(dst_hbm.at[pl.ds(off, CH)], didx)
            pltpu.async_copy(y_hbm.at[sidx], rows, sem).wait()
            pltpu.sync_copy(rows, acc.at[didx], add=True)

        plsc.subcore_barrier()
        pltpu.sync_copy(
            acc.at[pl.ds(sid * RPS, RPS)],
            out_hbm.at[cid, pl.ds(sid * RPS, RPS)],
        )

    return kernel


def _sc_deg_kernel():
    mesh = plsc.VectorSubcoreMesh(core_axis_name="c", subcore_axis_name="s")

    @functools.partial(
        pl.kernel,
        out_type=jax.ShapeDtypeStruct((2, NP, 16), jnp.float32),
        mesh=mesh,
        scratch_types=[
            pltpu.VMEM((CH,), jnp.int32),
            pltpu.VMEM((CH, 16), jnp.float32),
            pltpu.VMEM_SHARED((NP, 16), jnp.float32),
            pltpu.SemaphoreType.DMA,
        ],
        compiler_params=_SC_PARAMS,
    )
    def kernel(dst_hbm, out_hbm, didx, vals, acc, sem):
        cid = lax.axis_index("c")
        sid = lax.axis_index("s")
        wid = sid * 2 + cid
        base = wid * EPW

        @pl.loop(0, CH)
        def _(r):
            vals[r, pl.ds(0, 16)] = jnp.zeros((16,), jnp.float32)

        @pl.loop(0, RPS, step=CH)
        def _(j):
            pltpu.sync_copy(vals, acc.at[pl.ds(sid * RPS + j, CH)])

        @pl.loop(0, CH)
        def _(r):
            vals[r, pl.ds(0, 16)] = jnp.full((16,), 1.0, jnp.float32)

        plsc.subcore_barrier()

        @pl.loop(0, NCHUNK)
        def _(i):
            off = base + i * CH
            pltpu.sync_copy(dst_hbm.at[pl.ds(off, CH)], didx)
            pltpu.sync_copy(vals, acc.at[didx], add=True)

        plsc.subcore_barrier()
        pltpu.sync_copy(
            acc.at[pl.ds(sid * RPS, RPS)],
            out_hbm.at[cid, pl.ds(sid * RPS, RPS)],
        )

    return kernel


def _tc_layer1(x_pad, W1, degacc):
    def body(x_ref, w_ref, d_ref, y1_ref, dinv_ref):
        d = d_ref[...]
        deg = d[0, :, 0:1] + d[1, :, 0:1] + 1.0
        dinv = lax.rsqrt(deg)
        xw = jnp.dot(x_ref[...], w_ref[...], preferred_element_type=jnp.float32)
        y1_ref[...] = xw * dinv
        dinv_ref[...] = dinv

    return pl.pallas_call(
        body,
        out_shape=(
            jax.ShapeDtypeStruct((NP, D1), jnp.float32),
            jax.ShapeDtypeStruct((NP, 1), jnp.float32),
        ),
    )(x_pad, W1, degacc)


def _tc_layer2(agg1, y1, dinv, b1, W2p):
    def body(a_ref, y1_ref, dinv_ref, b1_ref, w2_ref, y2_ref):
        a = a_ref[...]
        dinv = dinv_ref[...]
        s = (a[0] + a[1] + y1_ref[...]) * dinv + b1_ref[...]
        h = jnp.maximum(s, 0.0)
        y2 = jnp.dot(h, w2_ref[...], preferred_element_type=jnp.float32) * dinv
        row = lax.broadcasted_iota(jnp.int32, (NP, D2), 0)
        y2_ref[...] = jnp.where(row < N, y2, 0.0)

    return pl.pallas_call(
        body,
        out_shape=jax.ShapeDtypeStruct((NP, D2), jnp.float32),
    )(agg1, y1, dinv, b1, W2p)


def _tc_final(agg2, y2, dinv, b2):
    def body(a_ref, y2_ref, dinv_ref, b2_ref, o_ref):
        a = a_ref[...]
        s = (a[0] + a[1] + y2_ref[...]) * dinv_ref[...]
        o = s[:N, :NCLS] + b2_ref[...]
        m = jnp.max(o, axis=1, keepdims=True)
        e = jnp.exp(o - m)
        lse = jnp.log(jnp.sum(e, axis=1, keepdims=True))
        o_ref[...] = o - m - lse

    return pl.pallas_call(
        body,
        out_shape=jax.ShapeDtypeStruct((N, NCLS), jnp.float32),
    )(agg2, y2, dinv, b2)


def kernel(x, edge_index, W1, b1, W2, b2):
    src = edge_index[0].astype(jnp.int32)
    dst = edge_index[1].astype(jnp.int32)
    npad = EP - E
    src_p = jnp.concatenate([src, jnp.full((npad,), N, jnp.int32)])
    dst_p = jnp.concatenate(
        [dst, N + (jnp.arange(npad, dtype=jnp.int32) % (NP - N))]
    )
    x_pad = jnp.pad(x, ((0, NP - N), (0, 0)))
    b1r = b1.reshape(1, D1)
    W2p = jnp.pad(W2, ((0, 0), (0, D2 - NCLS)))
    b2r = b2.reshape(1, NCLS)

    degacc = _sc_deg_kernel()(dst_p)
    y1, dinv = _tc_layer1(x_pad, W1, degacc)
    agg1 = _sc_agg_kernel(D1)(y1, src_p, dst_p)
    y2 = _tc_layer2(agg1, y1, dinv, b1r, W2p)
    agg2 = _sc_agg_kernel(D2)(y2, src_p, dst_p)
    return _tc_final(agg2, y2, dinv, b2r)


# speedup vs baseline: 27.3369x; 1.4763x over previous
import functools

import jax
import jax.numpy as jnp
from jax import lax
from jax.experimental import pallas as pl
from jax.experimental.pallas import tpu as pltpu
from jax.experimental.pallas import tpu_sc as plsc

_SC_PARAMS = pltpu.CompilerParams(use_tc_tiling_on_sc=False)

N = 10000
NP = 10240
E = 320000
CH = 128
NW = 32
NCHUNK = 80
EPW = NCHUNK * CH
EP = NW * EPW
NSUB = 16
RPS = NP // NSUB

D1 = 16
D2 = 48
NCLS = 40


def _sc_agg_kernel(D):
    mesh = plsc.VectorSubcoreMesh(core_axis_name="c", subcore_axis_name="s")

    @functools.partial(
        pl.kernel,
        out_type=jax.ShapeDtypeStruct((2, NP, D), jnp.float32),
        mesh=mesh,
        scratch_types=[
            pltpu.VMEM((NCHUNK, CH), jnp.int32),
            pltpu.VMEM((NCHUNK, CH), jnp.int32),
            pltpu.VMEM((CH, D), jnp.float32),
            pltpu.VMEM((CH, D), jnp.float32),
            pltpu.VMEM_SHARED((NP, D), jnp.float32),
            pltpu.SemaphoreType.DMA,
            pltpu.SemaphoreType.DMA,
        ],
        compiler_params=_SC_PARAMS,
    )
    def kernel(y_hbm, src_hbm, dst_hbm, out_hbm, sidx, didx, rows0, rows1,
               acc, sem0, sem1):
        cid = lax.axis_index("c")
        sid = lax.axis_index("s")
        wid = sid * 2 + cid
        cbase = wid * NCHUNK

        pltpu.sync_copy(src_hbm.at[pl.ds(cbase, NCHUNK)], sidx)
        pltpu.sync_copy(dst_hbm.at[pl.ds(cbase, NCHUNK)], didx)

        @pl.loop(0, CH)
        def _(r):
            @pl.loop(0, D, step=16)
            def _(c2):
                rows0[r, pl.ds(c2, 16)] = jnp.zeros((16,), jnp.float32)

        @pl.loop(0, RPS, step=CH)
        def _(j):
            pltpu.sync_copy(rows0, acc.at[pl.ds(sid * RPS + j, CH)])

        plsc.subcore_barrier()

        pltpu.async_copy(y_hbm.at[sidx.at[0]], rows0, sem0)
        pltpu.async_copy(y_hbm.at[sidx.at[1]], rows1, sem1)

        @pl.loop(0, NCHUNK - 2, step=2)
        def _(i):
            pltpu.make_async_copy(y_hbm.at[sidx.at[i]], rows0, sem0).wait()
            pltpu.sync_copy(rows0, acc.at[didx.at[i]], add=True)
            pltpu.async_copy(y_hbm.at[sidx.at[i + 2]], rows0, sem0)
            pltpu.make_async_copy(y_hbm.at[sidx.at[i + 1]], rows1, sem1).wait()
            pltpu.sync_copy(rows1, acc.at[didx.at[i + 1]], add=True)
            pltpu.async_copy(y_hbm.at[sidx.at[i + 3]], rows1, sem1)

        pltpu.make_async_copy(y_hbm.at[sidx.at[NCHUNK - 2]], rows0, sem0).wait()
        pltpu.sync_copy(rows0, acc.at[didx.at[NCHUNK - 2]], add=True)
        pltpu.make_async_copy(y_hbm.at[sidx.at[NCHUNK - 1]], rows1, sem1).wait()
        pltpu.sync_copy(rows1, acc.at[didx.at[NCHUNK - 1]], add=True)

        plsc.subcore_barrier()
        pltpu.sync_copy(
            acc.at[pl.ds(sid * RPS, RPS)],
            out_hbm.at[cid, pl.ds(sid * RPS, RPS)],
        )

    return kernel


def _sc_deg_kernel():
    mesh = plsc.VectorSubcoreMesh(core_axis_name="c", subcore_axis_name="s")

    @functools.partial(
        pl.kernel,
        out_type=jax.ShapeDtypeStruct((2, NP, 16), jnp.float32),
        mesh=mesh,
        scratch_types=[
            pltpu.VMEM((NCHUNK, CH), jnp.int32),
            pltpu.VMEM((CH, 16), jnp.float32),
            pltpu.VMEM_SHARED((NP, 16), jnp.float32),
            pltpu.SemaphoreType.DMA,
        ],
        compiler_params=_SC_PARAMS,
    )
    def kernel(dst_hbm, out_hbm, didx, vals, acc, sem):
        cid = lax.axis_index("c")
        sid = lax.axis_index("s")
        wid = sid * 2 + cid
        cbase = wid * NCHUNK

        pltpu.sync_copy(dst_hbm.at[pl.ds(cbase, NCHUNK)], didx)

        @pl.loop(0, CH)
        def _(r):
            vals[r, pl.ds(0, 16)] = jnp.zeros((16,), jnp.float32)

        @pl.loop(0, RPS, step=CH)
        def _(j):
            pltpu.sync_copy(vals, acc.at[pl.ds(sid * RPS + j, CH)])

        @pl.loop(0, CH)
        def _(r):
            vals[r, pl.ds(0, 16)] = jnp.full((16,), 1.0, jnp.float32)

        plsc.subcore_barrier()

        @pl.loop(0, NCHUNK)
        def _(i):
            pltpu.sync_copy(vals, acc.at[didx.at[i]], add=True)

        plsc.subcore_barrier()
        pltpu.sync_copy(
            acc.at[pl.ds(sid * RPS, RPS)],
            out_hbm.at[cid, pl.ds(sid * RPS, RPS)],
        )

    return kernel


def _tc_layer1(x_pad, W1, degacc):
    def body(x_ref, w_ref, d_ref, y1_ref, dinv_ref):
        d = d_ref[...]
        deg = d[0, :, 0:1] + d[1, :, 0:1] + 1.0
        dinv = lax.rsqrt(deg)
        xw = jnp.dot(x_ref[...], w_ref[...], preferred_element_type=jnp.float32)
        y1_ref[...] = xw * dinv
        dinv_ref[...] = dinv

    return pl.pallas_call(
        body,
        out_shape=(
            jax.ShapeDtypeStruct((NP, D1), jnp.float32),
            jax.ShapeDtypeStruct((NP, 1), jnp.float32),
        ),
    )(x_pad, W1, degacc)


def _tc_layer2(agg1, y1, dinv, b1, W2p):
    def body(a_ref, y1_ref, dinv_ref, b1_ref, w2_ref, y2_ref):
        a = a_ref[...]
        dinv = dinv_ref[...]
        s = (a[0] + a[1] + y1_ref[...]) * dinv + b1_ref[...]
        h = jnp.maximum(s, 0.0)
        y2 = jnp.dot(h, w2_ref[...], preferred_element_type=jnp.float32) * dinv
        row = lax.broadcasted_iota(jnp.int32, (NP, D2), 0)
        y2_ref[...] = jnp.where(row < N, y2, 0.0)

    return pl.pallas_call(
        body,
        out_shape=jax.ShapeDtypeStruct((NP, D2), jnp.float32),
    )(agg1, y1, dinv, b1, W2p)


def _tc_final(agg2, y2, dinv, b2):
    def body(a_ref, y2_ref, dinv_ref, b2_ref, o_ref):
        a = a_ref[...]
        s = (a[0] + a[1] + y2_ref[...]) * dinv_ref[...]
        o = s[:N, :NCLS] + b2_ref[...]
        m = jnp.max(o, axis=1, keepdims=True)
        e = jnp.exp(o - m)
        lse = jnp.log(jnp.sum(e, axis=1, keepdims=True))
        o_ref[...] = o - m - lse

    return pl.pallas_call(
        body,
        out_shape=jax.ShapeDtypeStruct((N, NCLS), jnp.float32),
    )(agg2, y2, dinv, b2)


def kernel(x, edge_index, W1, b1, W2, b2):
    src = edge_index[0].astype(jnp.int32)
    dst = edge_index[1].astype(jnp.int32)
    npad = EP - E
    src_p = jnp.concatenate([src, jnp.full((npad,), N, jnp.int32)])
    dst_p = jnp.concatenate(
        [dst, N + (jnp.arange(npad, dtype=jnp.int32) % (NP - N))]
    )
    src_p = src_p.reshape(NW * NCHUNK, CH)
    dst_p = dst_p.reshape(NW * NCHUNK, CH)
    x_pad = jnp.pad(x, ((0, NP - N), (0, 0)))
    b1r = b1.reshape(1, D1)
    W2p = jnp.pad(W2, ((0, 0), (0, D2 - NCLS)))
    b2r = b2.reshape(1, NCLS)

    degacc = _sc_deg_kernel()(dst_p)
    y1, dinv = _tc_layer1(x_pad, W1, degacc)
    agg1 = _sc_agg_kernel(D1)(y1, src_p, dst_p)
    y2 = _tc_layer2(agg1, y1, dinv, b1r, W2p)
    agg2 = _sc_agg_kernel(D2)(y2, src_p, dst_p)
    return _tc_final(agg2, y2, dinv, b2r)


# speedup vs baseline: 27.8248x; 1.0179x over previous
import functools

import jax
import jax.numpy as jnp
from jax import lax
from jax.experimental import pallas as pl
from jax.experimental.pallas import tpu as pltpu
from jax.experimental.pallas import tpu_sc as plsc

_SC_PARAMS = pltpu.CompilerParams(use_tc_tiling_on_sc=False)

N = 10000
NP = 10240
E = 320000
CH = 128
NW = 32
NCHUNK = 80
EPW = NCHUNK * CH
EP = NW * EPW
NSUB = 16
RPS = NP // NSUB
NBUF = 4

D1 = 16
D2 = 48
NCLS = 40


def _sc_agg_kernel(D):
    mesh = plsc.VectorSubcoreMesh(core_axis_name="c", subcore_axis_name="s")

    @functools.partial(
        pl.kernel,
        out_type=jax.ShapeDtypeStruct((2, NP, D), jnp.float32),
        mesh=mesh,
        scratch_types=[
            pltpu.VMEM((NCHUNK, CH), jnp.int32),
            pltpu.VMEM((NCHUNK, CH), jnp.int32),
            [pltpu.VMEM((CH, D), jnp.float32) for _ in range(NBUF)],
            pltpu.VMEM_SHARED((NP, D), jnp.float32),
            [pltpu.SemaphoreType.DMA for _ in range(NBUF)],
        ],
        compiler_params=_SC_PARAMS,
    )
    def kernel(y_hbm, src_hbm, dst_hbm, out_hbm, sidx, didx, rows, acc, sems):
        cid = lax.axis_index("c")
        sid = lax.axis_index("s")
        wid = sid * 2 + cid
        cbase = wid * NCHUNK

        pltpu.sync_copy(src_hbm.at[pl.ds(cbase, NCHUNK)], sidx)
        pltpu.sync_copy(dst_hbm.at[pl.ds(cbase, NCHUNK)], didx)

        @pl.loop(0, CH)
        def _(r):
            @pl.loop(0, D, step=16)
            def _(c2):
                rows[0][r, pl.ds(c2, 16)] = jnp.zeros((16,), jnp.float32)

        @pl.loop(0, RPS, step=CH)
        def _(j):
            pltpu.sync_copy(rows[0], acc.at[pl.ds(sid * RPS + j, CH)])

        plsc.subcore_barrier()

        for b in range(NBUF):
            pltpu.async_copy(y_hbm.at[sidx.at[b]], rows[b], sems[b])

        @pl.loop(0, NCHUNK - NBUF, step=NBUF)
        def _(i):
            for b in range(NBUF):
                pltpu.make_async_copy(
                    y_hbm.at[sidx.at[i + b]], rows[b], sems[b]).wait()
                pltpu.sync_copy(rows[b], acc.at[didx.at[i + b]], add=True)
                pltpu.async_copy(
                    y_hbm.at[sidx.at[i + b + NBUF]], rows[b], sems[b])

        for b in range(NBUF):
            c = NCHUNK - NBUF + b
            pltpu.make_async_copy(y_hbm.at[sidx.at[c]], rows[b], sems[b]).wait()
            pltpu.sync_copy(rows[b], acc.at[didx.at[c]], add=True)

        plsc.subcore_barrier()
        pltpu.sync_copy(
            acc.at[pl.ds(sid * RPS, RPS)],
            out_hbm.at[cid, pl.ds(sid * RPS, RPS)],
        )

    return kernel


def _sc_deg_kernel():
    mesh = plsc.VectorSubcoreMesh(core_axis_name="c", subcore_axis_name="s")

    @functools.partial(
        pl.kernel,
        out_type=jax.ShapeDtypeStruct((2, NP, 16), jnp.float32),
        mesh=mesh,
        scratch_types=[
            pltpu.VMEM((NCHUNK, CH), jnp.int32),
            pltpu.VMEM((CH, 16), jnp.float32),
            pltpu.VMEM_SHARED((NP, 16), jnp.float32),
            pltpu.SemaphoreType.DMA,
        ],
        compiler_params=_SC_PARAMS,
    )
    def kernel(dst_hbm, out_hbm, didx, vals, acc, sem):
        cid = lax.axis_index("c")
        sid = lax.axis_index("s")
        wid = sid * 2 + cid
        cbase = wid * NCHUNK

        pltpu.sync_copy(dst_hbm.at[pl.ds(cbase, NCHUNK)], didx)

        @pl.loop(0, CH)
        def _(r):
            vals[r, pl.ds(0, 16)] = jnp.zeros((16,), jnp.float32)

        @pl.loop(0, RPS, step=CH)
        def _(j):
            pltpu.sync_copy(vals, acc.at[pl.ds(sid * RPS + j, CH)])

        @pl.loop(0, CH)
        def _(r):
            vals[r, pl.ds(0, 16)] = jnp.full((16,), 1.0, jnp.float32)

        plsc.subcore_barrier()

        @pl.loop(0, NCHUNK)
        def _(i):
            pltpu.sync_copy(vals, acc.at[didx.at[i]], add=True)

        plsc.subcore_barrier()
        pltpu.sync_copy(
            acc.at[pl.ds(sid * RPS, RPS)],
            out_hbm.at[cid, pl.ds(sid * RPS, RPS)],
        )

    return kernel


def _tc_layer1(x_pad, W1, degacc):
    def body(x_ref, w_ref, d_ref, y1_ref, dinv_ref):
        d = d_ref[...]
        deg = d[0, :, 0:1] + d[1, :, 0:1] + 1.0
        dinv = lax.rsqrt(deg)
        xw = jnp.dot(x_ref[...], w_ref[...], preferred_element_type=jnp.float32)
        y1_ref[...] = xw * dinv
        dinv_ref[...] = dinv

    return pl.pallas_call(
        body,
        out_shape=(
            jax.ShapeDtypeStruct((NP, D1), jnp.float32),
            jax.ShapeDtypeStruct((NP, 1), jnp.float32),
        ),
    )(x_pad, W1, degacc)


def _tc_layer2(agg1, y1, dinv, b1, W2p):
    def body(a_ref, y1_ref, dinv_ref, b1_ref, w2_ref, y2_ref):
        a = a_ref[...]
        dinv = dinv_ref[...]
        s = (a[0] + a[1] + y1_ref[...]) * dinv + b1_ref[...]
        h = jnp.maximum(s, 0.0)
        y2 = jnp.dot(h, w2_ref[...], preferred_element_type=jnp.float32) * dinv
        row = lax.broadcasted_iota(jnp.int32, (NP, D2), 0)
        y2_ref[...] = jnp.where(row < N, y2, 0.0)

    return pl.pallas_call(
        body,
        out_shape=jax.ShapeDtypeStruct((NP, D2), jnp.float32),
    )(agg1, y1, dinv, b1, W2p)


def _tc_final(agg2, y2, dinv, b2):
    def body(a_ref, y2_ref, dinv_ref, b2_ref, o_ref):
        a = a_ref[...]
        s = (a[0] + a[1] + y2_ref[...]) * dinv_ref[...]
        o = s[:N, :NCLS] + b2_ref[...]
        m = jnp.max(o, axis=1, keepdims=True)
        e = jnp.exp(o - m)
        lse = jnp.log(jnp.sum(e, axis=1, keepdims=True))
        o_ref[...] = o - m - lse

    return pl.pallas_call(
        body,
        out_shape=jax.ShapeDtypeStruct((N, NCLS), jnp.float32),
    )(agg2, y2, dinv, b2)


def kernel(x, edge_index, W1, b1, W2, b2):
    src = edge_index[0].astype(jnp.int32)
    dst = edge_index[1].astype(jnp.int32)
    npad = EP - E
    src_p = jnp.concatenate([src, jnp.full((npad,), N, jnp.int32)])
    dst_p = jnp.concatenate(
        [dst, N + (jnp.arange(npad, dtype=jnp.int32) % (NP - N))]
    )
    src_p = src_p.reshape(NW * NCHUNK, CH)
    dst_p = dst_p.reshape(NW * NCHUNK, CH)
    x_pad = jnp.pad(x, ((0, NP - N), (0, 0)))
    b1r = b1.reshape(1, D1)
    W2p = jnp.pad(W2, ((0, 0), (0, D2 - NCLS)))
    b2r = b2.reshape(1, NCLS)

    degacc = _sc_deg_kernel()(dst_p)
    y1, dinv = _tc_layer1(x_pad, W1, degacc)
    agg1 = _sc_agg_kernel(D1)(y1, src_p, dst_p)
    y2 = _tc_layer2(agg1, y1, dinv, b1r, W2p)
    agg2 = _sc_agg_kernel(D2)(y2, src_p, dst_p)
    return _tc_final(agg2, y2, dinv, b2r)
